# baseline (speedup 1.0000x reference)
import functools

import jax
import jax.numpy as jnp
from jax import lax
from jax.experimental import pallas as pl
from jax.experimental.pallas import tpu as pltpu
from jax.experimental.pallas import tpu_sc as plsc

SEQ_LEN = 512
KMERS = 1025

_info = plsc.get_sparse_core_info()
_L = _info.num_lanes
_NS = _info.num_subcores
_NW = _NS
_CHUNK = SEQ_LEN // _NW

_mesh = plsc.VectorSubcoreMesh(
    core_axis_name="c", subcore_axis_name="s", num_cores=1)


@functools.partial(
    pl.kernel,
    mesh=_mesh,
    compiler_params=pltpu.CompilerParams(
        disable_bounds_checks=True,
        disable_semaphore_checks=True,
        skip_device_barrier=True,
        needs_layout_passes=False,
        use_tc_tiling_on_sc=False,
    ),
    out_type=jax.ShapeDtypeStruct((SEQ_LEN,), jnp.float32),
    scratch_types=[
        pltpu.VMEM((_CHUNK,), jnp.int32),
        pltpu.VMEM((_CHUNK,), jnp.float32),
        pltpu.VMEM((_CHUNK,), jnp.float32),
        pltpu.VMEM((_CHUNK,), jnp.float32),
        pltpu.SemaphoreType.DMA,
        pltpu.SemaphoreType.DMA,
        pltpu.SemaphoreType.DMA,
    ],
)
def _shmoof_sc(idx_hbm, table_hbm, site_hbm, out_hbm,
               idx_v, gathered_v, site_v, out_v, sem_idx, sem_site, sem_tab):
    base = lax.axis_index("s") * _CHUNK
    idx_cp = pltpu.async_copy(idx_hbm.at[pl.ds(base, _CHUNK)], idx_v, sem_idx)
    site_cp = pltpu.async_copy(site_hbm.at[pl.ds(base, _CHUNK)], site_v,
                               sem_site)
    idx_cp.wait()
    pltpu.async_copy(table_hbm.at[idx_v], gathered_v, sem_tab).wait()
    site_cp.wait()
    for j in range(_CHUNK // _L):
        s = pl.ds(j * _L, _L)
        out_v[s] = jnp.exp(gathered_v[s] + site_v[s])
    pltpu.sync_copy(out_v, out_hbm.at[pl.ds(base, _CHUNK)])


def kernel(encoded_parent, kmer_embedding, log_site_rates):
    table = kmer_embedding[:, 0]
    site = log_site_rates[:, 0]
    return _shmoof_sc(encoded_parent, table, site)


# speedup vs baseline: 1.0039x; 1.0039x over previous
import functools

import jax
import jax.numpy as jnp
from jax import lax
from jax.experimental import pallas as pl
from jax.experimental.pallas import tpu as pltpu
from jax.experimental.pallas import tpu_sc as plsc

SEQ_LEN = 512
KMERS = 1025

_info = plsc.get_sparse_core_info()
_L = _info.num_lanes
_NS = _info.num_subcores
_NW = _NS
_CHUNK = SEQ_LEN // _NW

_mesh = plsc.VectorSubcoreMesh(
    core_axis_name="c", subcore_axis_name="s", num_cores=1)


@functools.partial(
    pl.kernel,
    mesh=_mesh,
    out_type=jax.ShapeDtypeStruct((SEQ_LEN,), jnp.float32),
    scratch_types=[
        pltpu.VMEM((_CHUNK,), jnp.int32),
        pltpu.VMEM((_CHUNK,), jnp.float32),
        pltpu.VMEM((_CHUNK,), jnp.float32),
        pltpu.VMEM((_CHUNK,), jnp.float32),
        pltpu.SemaphoreType.DMA,
        pltpu.SemaphoreType.DMA,
        pltpu.SemaphoreType.DMA,
    ],
)
def _shmoof_sc(idx_hbm, table_hbm, site_hbm, out_hbm,
               idx_v, gathered_v, site_v, out_v, sem_idx, sem_site, sem_tab):
    base = lax.axis_index("s") * _CHUNK
    idx_cp = pltpu.async_copy(idx_hbm.at[pl.ds(base, _CHUNK)], idx_v, sem_idx)
    site_cp = pltpu.async_copy(site_hbm.at[pl.ds(base, _CHUNK)], site_v,
                               sem_site)
    idx_cp.wait()
    pltpu.async_copy(table_hbm.at[idx_v], gathered_v, sem_tab).wait()
    site_cp.wait()
    for j in range(_CHUNK // _L):
        s = pl.ds(j * _L, _L)
        out_v[s] = jnp.exp(gathered_v[s] + site_v[s])
    pltpu.sync_copy(out_v, out_hbm.at[pl.ds(base, _CHUNK)])


def kernel(encoded_parent, kmer_embedding, log_site_rates):
    table = kmer_embedding[:, 0]
    site = log_site_rates[:, 0]
    return _shmoof_sc(encoded_parent, table, site)
